# baseline (device time: 28652 ns/iter reference)
import jax
import jax.numpy as jnp
from jax import lax
from jax.experimental import pallas as pl
from jax.experimental.pallas import tpu as pltpu

M = 1024
N = 1024
HALF = N // 2


def _parity(a, j):
    return bin(a & j).count("1") % 2


def kernel(A, B):
    def body(a_ref, b_ref, out_ref, b16_ref, rbuf_ref, send_sems, recv_sems):
        p = lax.axis_index("i")
        bit0 = p % 2
        bit1 = (p // 2) % 2
        bit2 = (p // 4) % 2

        barrier = pltpu.get_barrier_semaphore()
        for mask in (3, 1, 4):
            pl.semaphore_signal(
                barrier,
                inc=1,
                device_id=(lax.bitwise_xor(p, mask),),
                device_id_type=pl.DeviceIdType.MESH,
            )

        row_parts = (
            (0, 384, 3, bit1, (1, 4, 5), (1, 4), (0, 48, 96), 384),
            (384, 384, 1, bit0 ^ bit1, (4, 3, 7), (4, 1), (144, 192, 240), 576),
            (768, 256, 4, bit2, (3, 1, 2), (2, 3), (288, 320, 352), 768),
        )
        bits_of = {1: bit0, 2: bit1, 3: bit0 ^ bit1, 4: bit2}

        parts = []
        for col in (0, HALF):
            for base, rows, m0, chi0, js, (a1, a2), soff, r1off in row_parts:
                half, quart, eighth = rows // 2, rows // 4, rows // 8
                keep0 = base + chi0 * half
                send0 = base + (1 - chi0) * half
                chi1 = bits_of[a1]
                chi2 = bits_of[a2]
                my8 = keep0 + chi1 * quart + chi2 * eighth
                peer8 = [
                    keep0
                    + (chi1 ^ _parity(a1, j)) * quart
                    + (chi2 ^ _parity(a2, j)) * eighth
                    for j in js
                ]
                parts.append(dict(
                    rows=rows, m0=m0, js=js, col=col,
                    keep0=keep0, send0=send0, half=half, eighth=eighth,
                    my8=my8, peer8=peer8, soff=soff, r1off=r1off,
                    sem0=11 * len(parts),
                ))

        inflight = {}

        def copy(pi, key, sem, src, dst, mask):
            pt = parts[pi]
            rdma = pltpu.make_async_remote_copy(
                src_ref=src,
                dst_ref=dst,
                send_sem=send_sems.at[pt["sem0"] + sem],
                recv_sem=recv_sems.at[pt["sem0"] + sem],
                device_id=(lax.bitwise_xor(p, mask),),
                device_id_type=pl.DeviceIdType.MESH,
            )
            rdma.start()
            inflight[(key, pi)] = rdma

        def rs1_start(pi):
            pt = parts[pi]
            sz, col = pt["half"], pt["col"]
            copy(
                pi, "rs1", 0,
                out_ref.at[pl.ds(pt["send0"], sz), pl.ds(col, HALF)],
                rbuf_ref.at[pl.ds(pt["r1off"], sz), pl.ds(col, HALF)],
                pt["m0"],
            )

        def rs1_finish_and_direct(pi):
            pt = parts[pi]
            sz, col = pt["half"], pt["col"]
            inflight.pop(("rs1", pi)).wait()
            out_ref[pl.ds(pt["keep0"], sz), pl.ds(col, HALF)] = (
                out_ref[pl.ds(pt["keep0"], sz), pl.ds(col, HALF)]
                + rbuf_ref[pl.ds(pt["r1off"], sz), pl.ds(col, HALF)]
            )
            e = pt["eighth"]
            for gi, j in enumerate(pt["js"]):
                copy(
                    pi, f"drs{gi}", 1 + gi,
                    out_ref.at[pl.ds(pt["peer8"][gi], e), pl.ds(col, HALF)],
                    rbuf_ref.at[pl.ds(pt["soff"][gi], e), pl.ds(col, HALF)],
                    j,
                )

        def direct_finish_and_ag(pi):
            pt = parts[pi]
            e, col = pt["eighth"], pt["col"]
            for gi in range(3):
                inflight.pop((f"drs{gi}", pi)).wait()
            out_ref[pl.ds(pt["my8"], e), pl.ds(col, HALF)] = jnp.maximum(
                out_ref[pl.ds(pt["my8"], e), pl.ds(col, HALF)]
                + rbuf_ref[pl.ds(pt["soff"][0], e), pl.ds(col, HALF)]
                + rbuf_ref[pl.ds(pt["soff"][1], e), pl.ds(col, HALF)]
                + rbuf_ref[pl.ds(pt["soff"][2], e), pl.ds(col, HALF)],
                0.0,
            )
            for gi, j in enumerate(pt["js"]):
                copy(
                    pi, f"dag{gi}", 4 + gi,
                    out_ref.at[pl.ds(pt["my8"], e), pl.ds(col, HALF)],
                    out_ref.at[pl.ds(pt["my8"], e), pl.ds(col, HALF)],
                    j,
                )
            copy(
                pi, "ag2m", 7,
                out_ref.at[pl.ds(pt["my8"], e), pl.ds(col, HALF)],
                out_ref.at[pl.ds(pt["my8"], e), pl.ds(col, HALF)],
                pt["m0"],
            )

        def dag_forward(pi, gi):
            pt = parts[pi]
            e, col = pt["eighth"], pt["col"]
            inflight.pop((f"dag{gi}", pi)).wait()
            copy(
                pi, f"ag2f{gi}", 8 + gi,
                out_ref.at[pl.ds(pt["peer8"][gi], e), pl.ds(col, HALF)],
                out_ref.at[pl.ds(pt["peer8"][gi], e), pl.ds(col, HALF)],
                pt["m0"],
            )

        b16_ref[:, :] = b_ref[:, :].astype(jnp.bfloat16)

        for pi, (lo, hi) in enumerate(((0, 384), (384, 768), (768, M))):
            out_ref[lo:hi, :] = jnp.dot(
                a_ref[lo:hi, :].astype(jnp.bfloat16),
                b16_ref[:, :],
                preferred_element_type=jnp.float32,
            ).astype(jnp.bfloat16)
            if pi == 0:
                pl.semaphore_wait(barrier, 3)
            rs1_start(pi)
            rs1_start(pi + 3)

        for pi in range(6):
            rs1_finish_and_direct(pi)
        for pi in range(6):
            direct_finish_and_ag(pi)
        for gi in range(3):
            for pi in range(6):
                dag_forward(pi, gi)
        for pi in range(6):
            inflight.pop(("ag2m", pi)).wait()
            for gi in range(3):
                inflight.pop((f"ag2f{gi}", pi)).wait()

    return pl.pallas_call(
        body,
        out_shape=jax.ShapeDtypeStruct((M, N), jnp.bfloat16),
        in_specs=[
            pl.BlockSpec(memory_space=pltpu.VMEM),
            pl.BlockSpec(memory_space=pltpu.VMEM),
        ],
        out_specs=pl.BlockSpec(memory_space=pltpu.VMEM),
        scratch_shapes=[
            pltpu.VMEM((512, N), jnp.bfloat16),
            pltpu.VMEM((896, N), jnp.bfloat16),
            pltpu.SemaphoreType.DMA((66,)),
            pltpu.SemaphoreType.DMA((66,)),
        ],
        compiler_params=pltpu.CompilerParams(collective_id=0),
    )(A, B)


# device time: 27000 ns/iter; 1.0612x vs baseline; 1.0612x over previous
import jax
import jax.numpy as jnp
from jax import lax
from jax.experimental import pallas as pl
from jax.experimental.pallas import tpu as pltpu

M = 1024
N = 1024
HALF = N // 2


def _parity(a, j):
    return bin(a & j).count("1") % 2


def kernel(A, B):
    def body(a_ref, b_ref, out_ref, b16_ref, rbuf_ref, send_sems, recv_sems):
        p = lax.axis_index("i")
        bit0 = p % 2
        bit1 = (p // 2) % 2
        bit2 = (p // 4) % 2

        barrier = pltpu.get_barrier_semaphore()
        for mask in (3, 1, 4):
            pl.semaphore_signal(
                barrier,
                inc=1,
                device_id=(lax.bitwise_xor(p, mask),),
                device_id_type=pl.DeviceIdType.MESH,
            )

        row_parts = (
            (0, 384, 3, bit1, (1, 4, 5), (1, 4), (0, 48, 96), 384),
            (384, 384, 1, bit0 ^ bit1, (4, 3, 7), (4, 1), (144, 192, 240), 576),
            (768, 256, 4, bit2, (3, 1, 2), (2, 3), (288, 320, 352), 768),
        )
        bits_of = {1: bit0, 2: bit1, 3: bit0 ^ bit1, 4: bit2}

        parts = []
        for col in (0, HALF):
            for base, rows, m0, chi0, js, (a1, a2), soff, r1off in row_parts:
                half, quart, eighth = rows // 2, rows // 4, rows // 8
                keep0 = base + chi0 * half
                send0 = base + (1 - chi0) * half
                chi1 = bits_of[a1]
                chi2 = bits_of[a2]
                my8 = keep0 + chi1 * quart + chi2 * eighth
                peer8 = [
                    keep0
                    + (chi1 ^ _parity(a1, j)) * quart
                    + (chi2 ^ _parity(a2, j)) * eighth
                    for j in js
                ]
                parts.append(dict(
                    rows=rows, m0=m0, js=js, col=col,
                    keep0=keep0, send0=send0, half=half, eighth=eighth,
                    my8=my8, peer8=peer8, soff=soff, r1off=r1off,
                    sem0=8 * len(parts),
                ))

        inflight = {}

        def copy(pi, key, sem, src, dst, mask):
            pt = parts[pi]
            rdma = pltpu.make_async_remote_copy(
                src_ref=src,
                dst_ref=dst,
                send_sem=send_sems.at[pt["sem0"] + sem],
                recv_sem=recv_sems.at[pt["sem0"] + sem],
                device_id=(lax.bitwise_xor(p, mask),),
                device_id_type=pl.DeviceIdType.MESH,
            )
            rdma.start()
            inflight[(key, pi)] = rdma

        def rs1_start(pi):
            pt = parts[pi]
            sz, col = pt["half"], pt["col"]
            copy(
                pi, "rs1", 0,
                out_ref.at[pl.ds(pt["send0"], sz), pl.ds(col, HALF)],
                rbuf_ref.at[pl.ds(pt["r1off"], sz), pl.ds(col, HALF)],
                pt["m0"],
            )

        def rs1_finish_and_direct(pi):
            pt = parts[pi]
            e, col = pt["eighth"], pt["col"]
            inflight.pop(("rs1", pi)).wait()
            for gi in (2, 0, 1):
                dst = pt["peer8"][gi]
                roff = pt["r1off"] + dst - pt["keep0"]
                out_ref[pl.ds(dst, e), pl.ds(col, HALF)] = (
                    out_ref[pl.ds(dst, e), pl.ds(col, HALF)]
                    + rbuf_ref[pl.ds(roff, e), pl.ds(col, HALF)]
                )
                copy(
                    pi, f"drs{gi}", 1 + gi,
                    out_ref.at[pl.ds(dst, e), pl.ds(col, HALF)],
                    rbuf_ref.at[pl.ds(pt["soff"][gi], e), pl.ds(col, HALF)],
                    pt["js"][gi],
                )
            my8 = pt["my8"]
            roff = pt["r1off"] + my8 - pt["keep0"]
            out_ref[pl.ds(my8, e), pl.ds(col, HALF)] = (
                out_ref[pl.ds(my8, e), pl.ds(col, HALF)]
                + rbuf_ref[pl.ds(roff, e), pl.ds(col, HALF)]
            )

        def direct_finish_and_ag(pi):
            pt = parts[pi]
            e, col = pt["eighth"], pt["col"]
            for gi in range(3):
                inflight.pop((f"drs{gi}", pi)).wait()
            out_ref[pl.ds(pt["my8"], e), pl.ds(col, HALF)] = jnp.maximum(
                out_ref[pl.ds(pt["my8"], e), pl.ds(col, HALF)]
                + rbuf_ref[pl.ds(pt["soff"][0], e), pl.ds(col, HALF)]
                + rbuf_ref[pl.ds(pt["soff"][1], e), pl.ds(col, HALF)]
                + rbuf_ref[pl.ds(pt["soff"][2], e), pl.ds(col, HALF)],
                0.0,
            )
            for gi in (2, 0, 1):
                copy(
                    pi, f"dag{gi}", 4 + gi,
                    out_ref.at[pl.ds(pt["my8"], e), pl.ds(col, HALF)],
                    out_ref.at[pl.ds(pt["my8"], e), pl.ds(col, HALF)],
                    pt["js"][gi],
                )

        def dag_finish_and_ag2(pi):
            pt = parts[pi]
            sz, col = pt["half"], pt["col"]
            for gi in range(3):
                inflight.pop((f"dag{gi}", pi)).wait()
            copy(
                pi, "ag2", 7,
                out_ref.at[pl.ds(pt["keep0"], sz), pl.ds(col, HALF)],
                out_ref.at[pl.ds(pt["keep0"], sz), pl.ds(col, HALF)],
                pt["m0"],
            )

        b16_ref[:, :] = b_ref[:, :].astype(jnp.bfloat16)

        def mm_half(pi, rows_base):
            pt = parts[pi]
            sz = pt["half"]
            out_ref[pl.ds(rows_base, sz), :] = jnp.dot(
                a_ref[pl.ds(rows_base, sz), :].astype(jnp.bfloat16),
                b16_ref[:, :],
                preferred_element_type=jnp.float32,
            ).astype(jnp.bfloat16)

        for pi in range(3):
            mm_half(pi, parts[pi]["send0"])
            if pi == 0:
                pl.semaphore_wait(barrier, 3)
            rs1_start(pi)
            rs1_start(pi + 3)
        for pi in range(3):
            mm_half(pi, parts[pi]["keep0"])

        for pi in range(6):
            rs1_finish_and_direct(pi)
        for pi in range(6):
            direct_finish_and_ag(pi)
        for pi in range(6):
            dag_finish_and_ag2(pi)
        for pi in range(6):
            inflight.pop(("ag2", pi)).wait()

    return pl.pallas_call(
        body,
        out_shape=jax.ShapeDtypeStruct((M, N), jnp.bfloat16),
        in_specs=[
            pl.BlockSpec(memory_space=pltpu.VMEM),
            pl.BlockSpec(memory_space=pltpu.VMEM),
        ],
        out_specs=pl.BlockSpec(memory_space=pltpu.VMEM),
        scratch_shapes=[
            pltpu.VMEM((512, N), jnp.bfloat16),
            pltpu.VMEM((896, N), jnp.bfloat16),
            pltpu.SemaphoreType.DMA((48,)),
            pltpu.SemaphoreType.DMA((48,)),
        ],
        compiler_params=pltpu.CompilerParams(collective_id=0),
    )(A, B)
